# baseline (speedup 1.0000x reference)
import jax
import jax.numpy as jnp
from jax import lax
from jax.experimental import pallas as pl
from jax.experimental.pallas import tpu as pltpu
from jax.experimental.pallas import tpu_sc as plsc

NBITS = 256
HIDDEN = 256
SPATIAL = 32
B = 128
C = 128
HW = SPATIAL * SPATIAL

NC = 2
NS = 16
NW = NC * NS
BPW = B // NW
LANES = 16
TCH = 32
SP_UNROLL = 16


def _sc_body(msg_hbm, lat_hbm, w_hbm, out_hbm,
             msg_v, idx_v, rows_v, acc_v, buf_v, sem):
    wid = lax.axis_index("s") * NC + lax.axis_index("c")
    lane = lax.iota(jnp.int32, LANES)
    nh = HIDDEN // LANES

    def _batch(j, _):
        b = wid * BPW + j

        pltpu.sync_copy(msg_hbm.at[b], msg_v)
        for t in range(NBITS // LANES):
            idx_v[pl.ds(t * LANES, LANES)] = (
                2 * (t * LANES + lane) + msg_v[pl.ds(t * LANES, LANES)])
        pltpu.async_copy(w_hbm.at[idx_v], rows_v, sem).wait()

        def _row(r, accs):
            return tuple(
                accs[t] + rows_v[r, pl.ds(t * LANES, LANES)]
                for t in range(nh))

        accs = tuple(jnp.zeros((LANES,), jnp.float32) for _ in range(nh))
        accs = lax.fori_loop(0, NBITS, _row, accs)
        for t in range(nh):
            acc_v[pl.ds(t * LANES, LANES)] = accs[t]

        for g in range(C // TCH):
            pltpu.sync_copy(lat_hbm.at[b, pl.ds(g * TCH, TCH)], buf_v)
            pltpu.sync_copy(buf_v, out_hbm.at[b, pl.ds(g * TCH, TCH)])

        n_sp = HW // LANES // SP_UNROLL

        def _group(g, _):
            for cl in range(TCH // LANES):
                vec = acc_v[pl.ds((g * (TCH // LANES) + cl) * LANES, LANES)]
                for k in range(LANES):
                    r = cl * LANES + k
                    splat = vec * 0.0 + vec[k]

                    def _sp(t, _, r=r, splat=splat):
                        for u in range(SP_UNROLL):
                            buf_v[r,
                                  pl.ds((t * SP_UNROLL + u) * LANES,
                                        LANES)] = splat
                        return 0

                    lax.fori_loop(0, n_sp, _sp, 0)
            pltpu.sync_copy(buf_v, out_hbm.at[b, pl.ds(C + g * TCH, TCH)])
            return 0

        lax.fori_loop(0, HIDDEN // TCH, _group, 0)
        return 0

    lax.fori_loop(0, BPW, _batch, 0)


def kernel(latents, msg, W):
    lat3 = latents.reshape(B, C, HW)
    mesh = plsc.VectorSubcoreMesh(core_axis_name="c", subcore_axis_name="s")
    out = pl.kernel(
        _sc_body,
        out_type=jax.ShapeDtypeStruct((B, C + HIDDEN, HW), jnp.float32),
        mesh=mesh,
        scratch_types=[
            pltpu.VMEM((NBITS,), jnp.int32),
            pltpu.VMEM((NBITS,), jnp.int32),
            pltpu.VMEM((NBITS, HIDDEN), jnp.float32),
            pltpu.VMEM((HIDDEN,), jnp.float32),
            pltpu.VMEM((TCH, HW), jnp.float32),
            pltpu.SemaphoreType.DMA,
        ],
    )(msg.astype(jnp.int32), lat3, W)
    return out.reshape(B, C + HIDDEN, SPATIAL, SPATIAL)
